# baseline (speedup 1.0000x reference)
import functools
import math

import jax
import jax.numpy as jnp
from jax import lax
from jax.experimental import pallas as pl
from jax.experimental.pallas import tpu as pltpu
from jax.experimental.pallas import tpu_sc as plsc

D_MODEL = 64
SCALE = math.sqrt(D_MODEL)
BTILE = 128
DSUB = 8
NBUF = 4


PACK_BLK = 8192


def _make_tc_pack(vocab: int, d: int):
    grid = (vocab + PACK_BLK - 1) // PACK_BLK

    def body(x_ref, o_ref):
        y = (x_ref[...].T * SCALE).reshape(PACK_BLK // 2, 2, d)
        o_ref[...] = jnp.concatenate([y[:, 0, :], y[:, 1, :]], axis=1)

    return pl.pallas_call(
        body,
        grid=(grid,),
        in_specs=[pl.BlockSpec((d, PACK_BLK), lambda i: (0, i))],
        out_specs=pl.BlockSpec((PACK_BLK // 2, 2 * d), lambda i: (i, 0)),
        out_shape=jax.ShapeDtypeStruct((vocab // 2, 2 * d), jnp.float32),
    )


def _make_sc_embed(seq: int, batch: int, vocab: int):
    info = plsc.get_sparse_core_info()
    nc, ns, nl = info.num_cores, info.num_subcores, info.num_lanes
    nw = nc * ns
    n_btiles = batch // BTILE
    assert n_btiles == nw and seq % NBUF == 0
    n_iter = seq // NBUF
    d_tiles = D_MODEL // DSUB

    mesh = plsc.VectorSubcoreMesh(core_axis_name="c", subcore_axis_name="s")

    @functools.partial(
        pl.kernel,
        out_type=jax.ShapeDtypeStruct(
            (seq, d_tiles, n_btiles, DSUB, BTILE), jnp.float32),
        mesh=mesh,
        scratch_types=(
            [pltpu.VMEM((seq, BTILE), jnp.int32)]
            + [pltpu.VMEM((BTILE, D_MODEL), jnp.float32)] * NBUF
            + [pltpu.VMEM((d_tiles, DSUB, BTILE + 1), jnp.float32)] * NBUF
            + [pltpu.SemaphoreType.DMA] * (2 * NBUF)
        ),
        compiler_params=pltpu.CompilerParams(
            use_tc_tiling_on_sc=False, needs_layout_passes=False,
            disable_bounds_checks=True),
    )
    def sc_embed(xt_hbm, table_hbm, out_hbm, idx_v, *bufs):
        gbufs = bufs[:NBUF]
        tbufs = bufs[NBUF:2 * NBUF]
        gsems = bufs[2 * NBUF:3 * NBUF]
        wsems = bufs[3 * NBUF:]
        w = lax.axis_index("s") * nc + lax.axis_index("c")

        pltpu.sync_copy(xt_hbm.at[:, pl.ds(w * BTILE, BTILE)], idx_v)

        def start_gather(s, gbuf, gsem):
            pltpu.async_copy(table_hbm.at[idx_v.at[s]], gbuf, gsem)

        def wait_gather(gbuf, gsem):
            pltpu.make_async_copy(
                table_hbm.at[idx_v.at[0]], gbuf, gsem).wait()

        def start_write(s, tbuf, wsem):
            pltpu.async_copy(
                tbuf.at[:, :, pl.ds(0, BTILE)], out_hbm.at[s, :, w], wsem)

        def wait_write(tbuf, wsem):
            pltpu.make_async_copy(
                tbuf.at[:, :, pl.ds(0, BTILE)],
                out_hbm.at[0, :, w], wsem).wait()

        _d = lax.iota(jnp.int32, nl)
        ti_vecs = [_d // DSUB + dd * (nl // DSUB)
                   for dd in range(D_MODEL // nl)]
        r_vec = _d % DSUB

        CPB = 8

        def transpose_scale(gbuf, tbuf):
            def per_cb(cb, carry):
                c0 = cb * CPB
                for i in range(CPB):
                    c = c0 + i
                    csp = jnp.full((nl,), c, jnp.int32)
                    for dd in range(D_MODEL // nl):
                        v = gbuf[c, pl.ds(dd * nl, nl)]
                        plsc.store_scatter(
                            tbuf, [ti_vecs[dd], r_vec, csp], v)
                return carry

            lax.fori_loop(0, BTILE // CPB, per_cb, 0)

        for p in range(NBUF):
            start_gather(p, gbufs[p], gsems[p])

        def body(k, carry):
            s_base = k * NBUF
            for p in range(NBUF):
                s = s_base + p
                wait_gather(gbufs[p], gsems[p])

                @pl.when(k > 0)
                def _(p=p):
                    wait_write(tbufs[p], wsems[p])

                transpose_scale(gbufs[p], tbufs[p])

                @pl.when(k < n_iter - 1)
                def _(p=p, s=s):
                    start_gather(s + NBUF, gbufs[p], gsems[p])

                start_write(s, tbufs[p], wsems[p])
            return carry

        lax.fori_loop(0, n_iter, body, 0)
        for p in range(NBUF):
            wait_write(tbufs[p], wsems[p])

    return sc_embed


def kernel(x, embed_weight):
    b, s = x.shape
    vocab, d = embed_weight.shape
    xt = x.T.astype(jnp.int32)
    packed = _make_tc_pack(vocab, d)(embed_weight.T)
    table_lin = packed.reshape(vocab, d)
    lin = _make_sc_embed(s, b, vocab)(xt, table_lin)
    return lax.reshape(lin, (b, s, d), dimensions=(2, 4, 0, 1, 3))


# speedup vs baseline: 1.0054x; 1.0054x over previous
import functools
import math

import jax
import jax.numpy as jnp
from jax import lax
from jax.experimental import pallas as pl
from jax.experimental.pallas import tpu as pltpu
from jax.experimental.pallas import tpu_sc as plsc

D_MODEL = 64
SCALE = math.sqrt(D_MODEL)
BTILE = 128
DSUB = 8
NBUF = 4


PACK_BLK = 16384


def _make_tc_pack(vocab: int, d: int):
    grid = (vocab + PACK_BLK - 1) // PACK_BLK

    def body(x_ref, o_ref):
        y = (x_ref[...].T * SCALE).reshape(PACK_BLK // 2, 2, d)
        o_ref[...] = jnp.concatenate([y[:, 0, :], y[:, 1, :]], axis=1)

    return pl.pallas_call(
        body,
        grid=(grid,),
        in_specs=[pl.BlockSpec((d, PACK_BLK), lambda i: (0, i))],
        out_specs=pl.BlockSpec((PACK_BLK // 2, 2 * d), lambda i: (i, 0)),
        out_shape=jax.ShapeDtypeStruct((vocab // 2, 2 * d), jnp.float32),
    )


def _make_sc_embed(seq: int, batch: int, vocab: int):
    info = plsc.get_sparse_core_info()
    nc, ns, nl = info.num_cores, info.num_subcores, info.num_lanes
    nw = nc * ns
    n_btiles = batch // BTILE
    assert n_btiles == nw and seq % NBUF == 0
    n_iter = seq // NBUF
    d_tiles = D_MODEL // DSUB

    mesh = plsc.VectorSubcoreMesh(core_axis_name="c", subcore_axis_name="s")

    @functools.partial(
        pl.kernel,
        out_type=jax.ShapeDtypeStruct(
            (seq, d_tiles, n_btiles, DSUB, BTILE), jnp.float32),
        mesh=mesh,
        scratch_types=(
            [pltpu.VMEM((seq, BTILE), jnp.int32)]
            + [pltpu.VMEM((BTILE, D_MODEL), jnp.float32)] * NBUF
            + [pltpu.VMEM((d_tiles, DSUB, BTILE + 1), jnp.float32)] * NBUF
            + [pltpu.SemaphoreType.DMA] * (2 * NBUF)
        ),
        compiler_params=pltpu.CompilerParams(
            use_tc_tiling_on_sc=False, needs_layout_passes=False,
            disable_bounds_checks=True),
    )
    def sc_embed(xt_hbm, table_hbm, out_hbm, idx_v, *bufs):
        gbufs = bufs[:NBUF]
        tbufs = bufs[NBUF:2 * NBUF]
        gsems = bufs[2 * NBUF:3 * NBUF]
        wsems = bufs[3 * NBUF:]
        w = lax.axis_index("s") * nc + lax.axis_index("c")

        pltpu.sync_copy(xt_hbm.at[:, pl.ds(w * BTILE, BTILE)], idx_v)

        def start_gather(s, gbuf, gsem):
            pltpu.async_copy(table_hbm.at[idx_v.at[s]], gbuf, gsem)

        def wait_gather(gbuf, gsem):
            pltpu.make_async_copy(
                table_hbm.at[idx_v.at[0]], gbuf, gsem).wait()

        def start_write(s, tbuf, wsem):
            pltpu.async_copy(
                tbuf.at[:, :, pl.ds(0, BTILE)], out_hbm.at[s, :, w], wsem)

        def wait_write(tbuf, wsem):
            pltpu.make_async_copy(
                tbuf.at[:, :, pl.ds(0, BTILE)],
                out_hbm.at[0, :, w], wsem).wait()

        _d = lax.iota(jnp.int32, nl)
        ti_vecs = [_d // DSUB + dd * (nl // DSUB)
                   for dd in range(D_MODEL // nl)]
        r_vec = _d % DSUB

        CPB = 8

        def transpose_scale(gbuf, tbuf):
            def per_cb(cb, carry):
                c0 = cb * CPB
                for i in range(CPB):
                    c = c0 + i
                    csp = jnp.full((nl,), c, jnp.int32)
                    for dd in range(D_MODEL // nl):
                        v = gbuf[c, pl.ds(dd * nl, nl)]
                        plsc.store_scatter(
                            tbuf, [ti_vecs[dd], r_vec, csp], v)
                return carry

            lax.fori_loop(0, BTILE // CPB, per_cb, 0)

        for p in range(NBUF):
            start_gather(p, gbufs[p], gsems[p])

        def body(k, carry):
            s_base = k * NBUF
            for p in range(NBUF):
                s = s_base + p
                wait_gather(gbufs[p], gsems[p])

                @pl.when(k > 0)
                def _(p=p):
                    wait_write(tbufs[p], wsems[p])

                transpose_scale(gbufs[p], tbufs[p])

                @pl.when(k < n_iter - 1)
                def _(p=p, s=s):
                    start_gather(s + NBUF, gbufs[p], gsems[p])

                start_write(s, tbufs[p], wsems[p])
            return carry

        lax.fori_loop(0, n_iter, body, 0)
        for p in range(NBUF):
            wait_write(tbufs[p], wsems[p])

    return sc_embed


def kernel(x, embed_weight):
    b, s = x.shape
    vocab, d = embed_weight.shape
    xt = x.T.astype(jnp.int32)
    packed = _make_tc_pack(vocab, d)(embed_weight.T)
    table_lin = packed.reshape(vocab, d)
    lin = _make_sc_embed(s, b, vocab)(xt, table_lin)
    return lax.reshape(lin, (b, s, d), dimensions=(2, 4, 0, 1, 3))
